# baseline (speedup 1.0000x reference)
import functools

import numpy as np

import jax
import jax.numpy as jnp
from jax import lax
from jax.experimental import pallas as pl
from jax.experimental.pallas import tpu as pltpu
from jax.experimental.pallas import tpu_sc as plsc

N = 10000
E = 320000
D = 128
NC = 2
NS = 16
NW = NC * NS
EPW = E // NW
C = 40
NCHUNK = EPW // C
RPT = 624
REM = N - NS * RPT
DW = 16

_PERM = np.empty(D, np.int64)
for _j in range(D // 32):
    for _k in range(16):
        _PERM[32 * _j + 2 * _k] = 32 * _j + _k
        _PERM[32 * _j + 2 * _k + 1] = 32 * _j + 16 + _k

_mesh = plsc.VectorSubcoreMesh(core_axis_name="c", subcore_axis_name="s")


@functools.partial(
    pl.kernel,
    mesh=_mesh,
    compiler_params=pltpu.CompilerParams(use_tc_tiling_on_sc=False,
                                         needs_layout_passes=False),
    out_type=(
        jax.ShapeDtypeStruct((NC, N, D), jnp.float32),
        jax.ShapeDtypeStruct((NC, N, DW), jnp.float32),
    ),
    scratch_types=[
        pltpu.VMEM_SHARED((N, D), jnp.float32),
        pltpu.VMEM_SHARED((N, DW), jnp.float32),
        pltpu.VMEM((EPW,), jnp.int32),
        [pltpu.VMEM((C,), jnp.int32) for _ in range(2)],
        [pltpu.VMEM((C, D // 2), jnp.int32) for _ in range(2)],
        [pltpu.VMEM((C, D), jnp.float32) for _ in range(2)],
        [pltpu.VMEM((C, D), jnp.float32) for _ in range(2)],
        pltpu.VMEM((C, DW), jnp.float32),
        [pltpu.SemaphoreType.DMA for _ in range(2)],
        [pltpu.SemaphoreType.DMA for _ in range(2)],
        [pltpu.SemaphoreType.DMA for _ in range(2)],
        [pltpu.SemaphoreType.DMA for _ in range(2)],
    ],
)
def _sc_segment_sum(src_hbm, dst_hbm, node_hbm, edge_hbm, z_acc_hbm, z_deg_hbm,
                    ones_hbm, acc_out, deg_out,
                    acc_sp, deg_sp, sidx_all, didx, gbuf, grows, erows, ones_v,
                    sem_i, sem_g, sem_e, scat_sem):
    cid = lax.axis_index("c")
    sid = lax.axis_index("s")
    wid = cid * NS + sid
    r0 = sid * RPT
    e0 = wid * EPW

    pltpu.sync_copy(src_hbm.at[pl.ds(e0, EPW)], sidx_all)
    pltpu.sync_copy(z_acc_hbm, acc_sp.at[pl.ds(r0, RPT)])
    pltpu.sync_copy(z_deg_hbm, deg_sp.at[pl.ds(r0, RPT)])

    @pl.when(sid == NS - 1)
    def _():
        pltpu.sync_copy(z_acc_hbm.at[pl.ds(0, REM)],
                        acc_sp.at[pl.ds(NS * RPT, REM)])
        pltpu.sync_copy(z_deg_hbm.at[pl.ds(0, REM)],
                        deg_sp.at[pl.ds(NS * RPT, REM)])

    pltpu.sync_copy(ones_hbm, ones_v)
    plsc.subcore_barrier()

    def issue(c, b):
        pltpu.async_copy(dst_hbm.at[pl.ds(e0 + c * C, C)], didx[b], sem_i[b])
        pltpu.async_copy(node_hbm.at[sidx_all.at[pl.ds(c * C, C)]], gbuf[b],
                         sem_g[b])
        pltpu.async_copy(edge_hbm.at[pl.ds(e0 + c * C, C)], erows[b],
                         sem_e[b])

    def complete_and_scatter(c, b):
        pltpu.make_async_copy(dst_hbm.at[pl.ds(e0 + c * C, C)], didx[b],
                              sem_i[b]).wait()
        pltpu.make_async_copy(node_hbm.at[sidx_all.at[pl.ds(c * C, C)]],
                              gbuf[b], sem_g[b]).wait()
        pltpu.make_async_copy(edge_hbm.at[pl.ds(e0 + c * C, C)], erows[b],
                              sem_e[b]).wait()

        def expand_row(r, carry):
            for j in range(D // 32):
                w = gbuf[b][r, pl.ds(16 * j, 16)]
                grows[b][r, pl.ds(32 * j, 16)] = plsc.bitcast(
                    w << 16, jnp.float32)
                grows[b][r, pl.ds(32 * j + 16, 16)] = plsc.bitcast(
                    w & jnp.int32(-65536), jnp.float32)
            return carry

        lax.fori_loop(0, C, expand_row, 0)
        pltpu.async_copy(grows[b], acc_sp.at[didx[b]], scat_sem[b], add=True)
        pltpu.async_copy(erows[b], acc_sp.at[didx[b]], scat_sem[b], add=True)
        pltpu.async_copy(ones_v, deg_sp.at[didx[b]], scat_sem[b], add=True)

    def drain_scatters(b):
        pltpu.make_async_copy(grows[b], acc_sp.at[didx[b]],
                              scat_sem[b]).wait()
        pltpu.make_async_copy(erows[b], acc_sp.at[didx[b]],
                              scat_sem[b]).wait()
        pltpu.make_async_copy(ones_v, deg_sp.at[didx[b]],
                              scat_sem[b]).wait()

    def pipe_pair(i, carry):
        for b in range(2):
            c = 2 * i + b

            @pl.when(jnp.logical_and(c >= 2, c <= NCHUNK + 1))
            def _():
                drain_scatters(b)

            @pl.when(c < NCHUNK)
            def _():
                issue(c, b)

            @pl.when(jnp.logical_and(c >= 1, c <= NCHUNK))
            def _():
                complete_and_scatter(c - 1, 1 - b)

        return carry

    lax.fori_loop(0, (NCHUNK + 2) // 2, pipe_pair, 0)
    plsc.subcore_barrier()

    pltpu.sync_copy(acc_sp.at[pl.ds(r0, RPT)], acc_out.at[cid, pl.ds(r0, RPT)])
    pltpu.sync_copy(deg_sp.at[pl.ds(r0, RPT)], deg_out.at[cid, pl.ds(r0, RPT)])

    @pl.when(sid == NS - 1)
    def _():
        pltpu.sync_copy(acc_sp.at[pl.ds(NS * RPT, REM)],
                        acc_out.at[cid, pl.ds(NS * RPT, REM)])
        pltpu.sync_copy(deg_sp.at[pl.ds(NS * RPT, REM)],
                        deg_out.at[cid, pl.ds(NS * RPT, REM)])


BLK = 1000


def _combine_body(eps_ref, node_ref, acc_ref, deg_ref, out_ref):
    deg = deg_ref[0, :, 0:1] + deg_ref[1, :, 0:1]
    neigh = (acc_ref[0] + acc_ref[1]) / jnp.maximum(deg, 1.0)
    out_ref[...] = (1.0 + eps_ref[0]) * node_ref[...] + neigh


_combine = pl.pallas_call(
    _combine_body,
    grid=(N // BLK,),
    in_specs=[
        pl.BlockSpec(memory_space=pltpu.SMEM),
        pl.BlockSpec((BLK, D), lambda i: (i, 0)),
        pl.BlockSpec((NC, BLK, D), lambda i: (0, i, 0)),
        pl.BlockSpec((NC, BLK, DW), lambda i: (0, i, 0)),
    ],
    out_specs=pl.BlockSpec((BLK, D), lambda i: (i, 0)),
    out_shape=jax.ShapeDtypeStruct((N, D), jnp.float32),
)


@jax.jit
def kernel(node_feat, edge_feat, edge_index, eps):
    src = edge_index[0]
    dst = edge_index[1]
    node_packed = lax.bitcast_convert_type(
        node_feat[:, _PERM].astype(jnp.bfloat16).reshape(N, D // 2, 2),
        jnp.int32)
    z_acc = jnp.zeros((RPT, D), jnp.float32)
    z_deg = jnp.zeros((RPT, DW), jnp.float32)
    ones = jnp.ones((C, DW), jnp.float32)
    acc, deg = _sc_segment_sum(src, dst, node_packed, edge_feat, z_acc, z_deg,
                               ones)
    return _combine(eps, node_feat, acc, deg)


# speedup vs baseline: 1.5780x; 1.5780x over previous
import functools

import jax
import jax.numpy as jnp
from jax import lax
from jax.experimental import pallas as pl
from jax.experimental.pallas import tpu as pltpu
from jax.experimental.pallas import tpu_sc as plsc

N = 10000
E = 320000
D = 128
NC = 2
NS = 16
NW = NC * NS
EPW = E // NW
C = 48
NCHUNK = EPW // C
TAIL = EPW - NCHUNK * C
RPT = 624
REM = N - NS * RPT
DW = 16

_mesh = plsc.VectorSubcoreMesh(core_axis_name="c", subcore_axis_name="s")


@functools.partial(
    pl.kernel,
    mesh=_mesh,
    compiler_params=pltpu.CompilerParams(use_tc_tiling_on_sc=False),
    out_type=(
        jax.ShapeDtypeStruct((NC, N, D), jnp.float32),
        jax.ShapeDtypeStruct((NC, N, DW), jnp.float32),
    ),
    scratch_types=[
        pltpu.VMEM_SHARED((N, D), jnp.float32),
        pltpu.VMEM_SHARED((N, DW), jnp.float32),
        pltpu.VMEM((EPW,), jnp.int32),
        [pltpu.VMEM((C,), jnp.int32) for _ in range(2)],
        pltpu.VMEM((TAIL,), jnp.int32),
        pltpu.VMEM((TAIL,), jnp.int32),
        [pltpu.VMEM((C, D), jnp.float32) for _ in range(2)],
        [pltpu.VMEM((C, D), jnp.float32) for _ in range(2)],
        pltpu.VMEM((C, DW), jnp.float32),
        [pltpu.SemaphoreType.DMA for _ in range(2)],
        [pltpu.SemaphoreType.DMA for _ in range(2)],
        [pltpu.SemaphoreType.DMA for _ in range(2)],
        [pltpu.SemaphoreType.DMA for _ in range(2)],
    ],
)
def _sc_segment_sum(src_hbm, dst_hbm, node_hbm, edge_hbm, z_acc_hbm, z_deg_hbm,
                    ones_hbm, acc_out, deg_out,
                    acc_sp, deg_sp, sidx_all, didx, tidx, tsidx, grows, erows,
                    ones_v, sem_i, sem_g, sem_e, scat_sem):
    cid = lax.axis_index("c")
    sid = lax.axis_index("s")
    wid = cid * NS + sid
    r0 = sid * RPT
    e0 = wid * EPW

    pltpu.sync_copy(src_hbm.at[pl.ds(e0, EPW)], sidx_all)
    pltpu.sync_copy(z_acc_hbm, acc_sp.at[pl.ds(r0, RPT)])
    pltpu.sync_copy(z_deg_hbm, deg_sp.at[pl.ds(r0, RPT)])

    @pl.when(sid == NS - 1)
    def _():
        pltpu.sync_copy(z_acc_hbm.at[pl.ds(0, REM)],
                        acc_sp.at[pl.ds(NS * RPT, REM)])
        pltpu.sync_copy(z_deg_hbm.at[pl.ds(0, REM)],
                        deg_sp.at[pl.ds(NS * RPT, REM)])

    pltpu.sync_copy(ones_hbm, ones_v)
    plsc.subcore_barrier()

    pltpu.sync_copy(dst_hbm.at[pl.ds(e0 + NCHUNK * C, TAIL)], tidx)
    pltpu.sync_copy(src_hbm.at[pl.ds(e0 + NCHUNK * C, TAIL)], tsidx)
    cp = pltpu.async_copy(node_hbm.at[tsidx], grows[0].at[pl.ds(0, TAIL)],
                          sem_g[0])
    pltpu.sync_copy(edge_hbm.at[pl.ds(e0 + NCHUNK * C, TAIL)],
                    erows[0].at[pl.ds(0, TAIL)])
    cp.wait()
    pltpu.sync_copy(grows[0].at[pl.ds(0, TAIL)], acc_sp.at[tidx], add=True)
    pltpu.sync_copy(erows[0].at[pl.ds(0, TAIL)], acc_sp.at[tidx], add=True)
    pltpu.sync_copy(ones_v.at[pl.ds(0, TAIL)], deg_sp.at[tidx], add=True)

    def issue(c, b):
        pltpu.async_copy(dst_hbm.at[pl.ds(e0 + c * C, C)], didx[b], sem_i[b])
        pltpu.async_copy(node_hbm.at[sidx_all.at[pl.ds(c * C, C)]], grows[b],
                         sem_g[b])
        pltpu.async_copy(edge_hbm.at[pl.ds(e0 + c * C, C)], erows[b],
                         sem_e[b])

    def complete_and_scatter(c, b):
        pltpu.make_async_copy(dst_hbm.at[pl.ds(e0 + c * C, C)], didx[b],
                              sem_i[b]).wait()
        pltpu.make_async_copy(node_hbm.at[sidx_all.at[pl.ds(c * C, C)]],
                              grows[b], sem_g[b]).wait()
        pltpu.make_async_copy(edge_hbm.at[pl.ds(e0 + c * C, C)], erows[b],
                              sem_e[b]).wait()
        pltpu.async_copy(grows[b], acc_sp.at[didx[b]], scat_sem[b], add=True)
        pltpu.async_copy(erows[b], acc_sp.at[didx[b]], scat_sem[b], add=True)
        pltpu.async_copy(ones_v, deg_sp.at[didx[b]], scat_sem[b], add=True)

    def drain_scatters(b):
        pltpu.make_async_copy(grows[b], acc_sp.at[didx[b]],
                              scat_sem[b]).wait()
        pltpu.make_async_copy(erows[b], acc_sp.at[didx[b]],
                              scat_sem[b]).wait()
        pltpu.make_async_copy(ones_v, deg_sp.at[didx[b]],
                              scat_sem[b]).wait()

    def pipe_pair(i, carry):
        for b in range(2):
            c = 2 * i + b

            @pl.when(jnp.logical_and(c >= 2, c <= NCHUNK + 1))
            def _():
                drain_scatters(b)

            @pl.when(c < NCHUNK)
            def _():
                issue(c, b)

            @pl.when(jnp.logical_and(c >= 1, c <= NCHUNK))
            def _():
                complete_and_scatter(c - 1, 1 - b)

        return carry

    lax.fori_loop(0, (NCHUNK + 2) // 2, pipe_pair, 0)
    plsc.subcore_barrier()

    pltpu.sync_copy(acc_sp.at[pl.ds(r0, RPT)], acc_out.at[cid, pl.ds(r0, RPT)])
    pltpu.sync_copy(deg_sp.at[pl.ds(r0, RPT)], deg_out.at[cid, pl.ds(r0, RPT)])

    @pl.when(sid == NS - 1)
    def _():
        pltpu.sync_copy(acc_sp.at[pl.ds(NS * RPT, REM)],
                        acc_out.at[cid, pl.ds(NS * RPT, REM)])
        pltpu.sync_copy(deg_sp.at[pl.ds(NS * RPT, REM)],
                        deg_out.at[cid, pl.ds(NS * RPT, REM)])


BLK = 1000


def _combine_body(eps_ref, node_ref, acc_ref, deg_ref, out_ref):
    deg = deg_ref[0, :, 0:1] + deg_ref[1, :, 0:1]
    neigh = (acc_ref[0] + acc_ref[1]) / jnp.maximum(deg, 1.0)
    out_ref[...] = (1.0 + eps_ref[0]) * node_ref[...] + neigh


_combine = pl.pallas_call(
    _combine_body,
    grid=(N // BLK,),
    in_specs=[
        pl.BlockSpec(memory_space=pltpu.SMEM),
        pl.BlockSpec((BLK, D), lambda i: (i, 0)),
        pl.BlockSpec((NC, BLK, D), lambda i: (0, i, 0)),
        pl.BlockSpec((NC, BLK, DW), lambda i: (0, i, 0)),
    ],
    out_specs=pl.BlockSpec((BLK, D), lambda i: (i, 0)),
    out_shape=jax.ShapeDtypeStruct((N, D), jnp.float32),
)


@jax.jit
def kernel(node_feat, edge_feat, edge_index, eps):
    src = edge_index[0]
    dst = edge_index[1]
    z_acc = jnp.zeros((RPT, D), jnp.float32)
    z_deg = jnp.zeros((RPT, DW), jnp.float32)
    ones = jnp.ones((C, DW), jnp.float32)
    acc, deg = _sc_segment_sum(src, dst, node_feat, edge_feat, z_acc, z_deg,
                               ones)
    return _combine(eps, node_feat, acc, deg)
